# Initial kernel scaffold; baseline (speedup 1.0000x reference)
import jax
import jax.numpy as jnp
from jax.experimental import pallas as pl


def kernel(x, edge_index, W1, att_src1, att_dst1, b1, W2, att_src2, att_dst2, b2):
    raise NotImplementedError("write your pallas kernel here")


# speedup vs baseline: 14.1913x; 14.1913x over previous
import functools

import jax
import jax.numpy as jnp
from jax import lax
from jax.experimental import pallas as pl
from jax.experimental.pallas import tpu as pltpu
from jax.experimental.pallas import tpu_sc as plsc

NC = 2
NS = 16
LANES = 16

F32 = jnp.float32
I32 = jnp.int32


def _lrelu(v):
  return jnp.where(v >= 0, v, 0.2 * v)


def _sds(shape, dtype):
  return jax.ShapeDtypeStruct(shape, dtype)


def _tc1(xp, W1, Asrc, Adst, *, npad, rblk, heads, interpret):
  nblk = npad // rblk
  in_ch = xp.shape[1]
  hidtot = W1.shape[1]

  def body(x_ref, w_ref, as_ref, ad_ref, h_ref, asrc_ref, adst_ref, smax_ref):
    h = jnp.dot(x_ref[...], w_ref[...], preferred_element_type=F32)
    h_ref[...] = h
    a_s = jnp.dot(h, as_ref[...], preferred_element_type=F32)
    a_d = jnp.dot(h, ad_ref[...], preferred_element_type=F32)
    asrc_ref[...] = a_s
    adst_ref[...] = a_d
    bm = jnp.max(a_s, axis=0, keepdims=True)

    @pl.when(pl.program_id(0) == 0)
    def _():
      smax_ref[...] = bm

    @pl.when(pl.program_id(0) > 0)
    def _():
      smax_ref[...] = jnp.maximum(smax_ref[...], bm)

  return pl.pallas_call(
      body,
      grid=(nblk,),
      in_specs=[
          pl.BlockSpec((rblk, in_ch), lambda i: (i, 0)),
          pl.BlockSpec((in_ch, hidtot), lambda i: (0, 0)),
          pl.BlockSpec((hidtot, LANES), lambda i: (0, 0)),
          pl.BlockSpec((hidtot, LANES), lambda i: (0, 0)),
      ],
      out_specs=[
          pl.BlockSpec((rblk, hidtot), lambda i: (i, 0)),
          pl.BlockSpec((rblk, LANES), lambda i: (i, 0)),
          pl.BlockSpec((rblk, LANES), lambda i: (i, 0)),
          pl.BlockSpec((1, LANES), lambda i: (0, 0)),
      ],
      out_shape=[
          _sds((npad, hidtot), F32),
          _sds((npad, LANES), F32),
          _sds((npad, LANES), F32),
          _sds((1, LANES), F32),
      ],
      interpret=interpret,
  )(xp, W1, Asrc, Adst)


def _sca(src, dst, asrc, adst, smax, z16, *, npad, epad, bsz, heads,
         interpret):
  nw = NC * NS
  chunk = epad // nw
  nbatch = chunk // bsz
  srows = npad // NS
  mesh = plsc.VectorSubcoreMesh(
      core_axis_name="c", subcore_axis_name="s", num_cores=NC, num_subcores=NS)

  @functools.partial(
      pl.kernel,
      out_type=(_sds((heads, epad), F32), _sds((NC, npad, LANES), F32)),
      mesh=mesh,
      compiler_params=pltpu.CompilerParams(needs_layout_passes=False, use_tc_tiling_on_sc=False),
      scratch_types=[
          pltpu.VMEM((bsz,), I32),
          pltpu.VMEM((bsz,), I32),
          pltpu.VMEM((bsz, LANES), F32),
          pltpu.VMEM((bsz, LANES), F32),
          pltpu.VMEM((bsz, LANES), F32),
          pltpu.VMEM((bsz * LANES,), F32),
          pltpu.VMEM((heads, bsz), F32),
          pltpu.VMEM((LANES,), F32),
          pltpu.VMEM_SHARED((npad, LANES), F32),
          pltpu.SemaphoreType.DMA,
          pltpu.SemaphoreType.DMA,
      ],
      interpret=interpret,
  )
  def k(src_h, dst_h, asrc_h, adst_h, smax_h, z16_h, ext_h, den_h,
        sidv, didv, sbuf, dbuf, aos, aosf, soa, smv, acc, sem1, sem2):
    c = lax.axis_index("c")
    s = lax.axis_index("s")
    w = s * NC + c
    pltpu.sync_copy(smax_h, smv)
    pltpu.sync_copy(z16_h.at[pl.ds(s * srows, srows)],
                    acc.at[pl.ds(s * srows, srows)])
    plsc.subcore_barrier()

    @pl.loop(0, nbatch)
    def _(t):
      off = w * chunk + t * bsz
      pltpu.sync_copy(src_h.at[pl.ds(off, bsz)], sidv)
      pltpu.sync_copy(dst_h.at[pl.ds(off, bsz)], didv)
      d1 = pltpu.async_copy(asrc_h.at[sidv], sbuf, sem1)
      d2 = pltpu.async_copy(adst_h.at[didv], dbuf, sem2)
      d1.wait()
      d2.wait()
      smaxv = smv[...]
      for r in range(bsz):
        sv = sbuf[r, :]
        dv = dbuf[r, :]
        al = _lrelu(sv + dv)
        cb = _lrelu(smaxv + dv)
        ev = jnp.exp(al - cb)
        aos[r, :] = ev
        aosf[pl.ds(r * LANES, LANES)] = ev
      lane = jax.lax.iota(I32, LANES)
      for g in range(bsz // LANES):
        ridx = (g * LANES + lane) * LANES
        for kk in range(heads):
          col = plsc.load_gather(aosf, [ridx + kk])
          soa[kk, pl.ds(g * LANES, LANES)] = col
      pltpu.sync_copy(soa, ext_h.at[:, pl.ds(off, bsz)])
      pltpu.sync_copy(aos, acc.at[didv], add=True)

    plsc.subcore_barrier()
    pltpu.sync_copy(acc.at[pl.ds(s * srows, srows)],
                    den_h.at[c, pl.ds(s * srows, srows)])

  return k(src, dst, asrc, adst, smax, z16)


def _scb(idx16, dst, ext, h2d, z32, *, npad, epad, bsz, interpret):
  chunk = epad // NS
  nbatch = chunk // bsz
  srows = npad // NS
  npass = 8
  mesh = plsc.VectorSubcoreMesh(
      core_axis_name="c", subcore_axis_name="s", num_cores=NC, num_subcores=NS)

  @functools.partial(
      pl.kernel,
      out_type=_sds((2 * npass, npad, 32), F32),
      mesh=mesh,
      compiler_params=pltpu.CompilerParams(needs_layout_passes=False, use_tc_tiling_on_sc=False),
      scratch_types=[
          pltpu.VMEM((bsz,), I32),
          pltpu.VMEM((bsz,), I32),
          pltpu.VMEM((bsz,), F32),
          pltpu.VMEM((bsz, 32), F32),
          pltpu.VMEM_SHARED((npad, 32), F32),
          pltpu.SemaphoreType.DMA,
      ],
      interpret=interpret,
  )
  def k(idx16_h, dst_h, ext_h, h2d_h, z32_h, u_h,
        didv, idxv, exv, hbuf, acc, sem):
    c = lax.axis_index("c")
    s = lax.axis_index("s")

    @pl.loop(0, npass)
    def _(p):
      j = c * npass + p
      head = j // 2
      pltpu.sync_copy(z32_h.at[pl.ds(s * srows, srows)],
                      acc.at[pl.ds(s * srows, srows)])
      plsc.subcore_barrier()

      @pl.loop(0, nbatch)
      def _(t):
        off = s * chunk + t * bsz
        pltpu.sync_copy(idx16_h.at[j, pl.ds(off, bsz)], idxv)
        pltpu.sync_copy(dst_h.at[pl.ds(off, bsz)], didv)
        pltpu.sync_copy(ext_h.at[head, pl.ds(off, bsz)], exv)
        pltpu.async_copy(h2d_h.at[idxv], hbuf, sem).wait()
        for g in range(bsz // LANES):
          evec = exv[pl.ds(g * LANES, LANES)]
          for rr in range(LANES):
            r = g * LANES + rr
            ev = evec.at[jnp.full((LANES,), rr, I32)].get(
                mode="promise_in_bounds")
            hbuf[r, 0:16] = hbuf[r, 0:16] * ev
            hbuf[r, 16:32] = hbuf[r, 16:32] * ev
        pltpu.sync_copy(hbuf, acc.at[didv], add=True)

      plsc.subcore_barrier()
      pltpu.sync_copy(acc.at[pl.ds(s * srows, srows)],
                      u_h.at[j, pl.ds(s * srows, srows)])
      plsc.subcore_barrier()

  return k(idx16, dst, ext, h2d, z32)


def _tcc(U, denp, b1r, W2r, *, n, npad, rblk, heads, interpret):
  nblk = npad // rblk

  def body(u_ref, dp_ref, b1_ref, w2_ref, h2_ref, mm_ref):
    den = dp_ref[0, :, 0:heads] + dp_ref[1, :, 0:heads] + 1e-16
    acc = jnp.zeros((rblk, 1), F32)
    for j in range(16):
      u = u_ref[j]
      dj = den[:, j // 2][:, None]
      hj = u / dj + b1_ref[j][None, :]
      hj = jnp.where(hj > 0, hj, jnp.exp(hj) - 1.0)
      acc = acc + jnp.dot(hj, w2_ref[j][:, None], preferred_element_type=F32)
    h2_ref[...] = acc
    rows = pl.program_id(0) * rblk + lax.broadcasted_iota(I32, (rblk, 1), 0)
    valid = rows < n
    hx = jnp.max(jnp.where(valid, acc, -jnp.inf)).reshape(1, 1)
    hn = jnp.min(jnp.where(valid, acc, jnp.inf)).reshape(1, 1)
    bm = jnp.concatenate([hn, hx], axis=1)

    @pl.when(pl.program_id(0) == 0)
    def _():
      mm_ref[...] = bm

    @pl.when(pl.program_id(0) > 0)
    def _():
      prev = mm_ref[...]
      mm_ref[...] = jnp.concatenate(
          [jnp.minimum(prev[:, 0:1], bm[:, 0:1]),
           jnp.maximum(prev[:, 1:2], bm[:, 1:2])], axis=1)

  return pl.pallas_call(
      body,
      grid=(nblk,),
      in_specs=[
          pl.BlockSpec((16, rblk, 32), lambda i: (0, i, 0)),
          pl.BlockSpec((NC, rblk, LANES), lambda i: (0, i, 0)),
          pl.BlockSpec((16, 32), lambda i: (0, 0)),
          pl.BlockSpec((16, 32), lambda i: (0, 0)),
      ],
      out_specs=[
          pl.BlockSpec((rblk, 1), lambda i: (i, 0)),
          pl.BlockSpec((1, 2), lambda i: (0, 0)),
      ],
      out_shape=[_sds((npad, 1), F32), _sds((1, 2), F32)],
      interpret=interpret,
  )(U, denp, b1r, W2r)


def _scd(src, dst, h2flat, params, z16, *, npad, epad, bsz, interpret):
  nw = NC * NS
  chunk = epad // nw
  nbatch = chunk // bsz
  srows = npad // NS
  mesh = plsc.VectorSubcoreMesh(
      core_axis_name="c", subcore_axis_name="s", num_cores=NC, num_subcores=NS)

  @functools.partial(
      pl.kernel,
      out_type=_sds((NC, npad, LANES), F32),
      mesh=mesh,
      compiler_params=pltpu.CompilerParams(needs_layout_passes=False, use_tc_tiling_on_sc=False),
      scratch_types=[
          pltpu.VMEM((npad,), F32),
          pltpu.VMEM((bsz,), I32),
          pltpu.VMEM((bsz,), I32),
          pltpu.VMEM((bsz, LANES), F32),
          pltpu.VMEM((LANES,), F32),
          pltpu.VMEM_SHARED((npad, LANES), F32),
      ],
      interpret=interpret,
  )
  def k(src_h, dst_h, h2_h, par_h, z16_h, out_h,
        h2v, sidv, didv, stage, pv, acc):
    c = lax.axis_index("c")
    s = lax.axis_index("s")
    w = s * NC + c
    pltpu.sync_copy(h2_h, h2v)
    pltpu.sync_copy(par_h, pv)
    pltpu.sync_copy(z16_h.at[pl.ds(s * srows, srows)],
                    acc.at[pl.ds(s * srows, srows)])
    plsc.subcore_barrier()
    lane = jax.lax.iota(I32, LANES)
    pvv = pv[...]
    take = lambda v, i: v.at[jnp.full((LANES,), i, I32)].get(
        mode="promise_in_bounds")
    cs = take(pvv, 0)
    cd = take(pvv, 1)
    mnv = take(pvv, 2)
    mxv = take(pvv, 3)
    s2max = jnp.maximum(cs * mxv, cs * mnv)
    zv = jnp.zeros((LANES,), F32)

    @pl.loop(0, nbatch)
    def _(t):
      off = w * chunk + t * bsz
      pltpu.sync_copy(src_h.at[pl.ds(off, bsz)], sidv)
      pltpu.sync_copy(dst_h.at[pl.ds(off, bsz)], didv)
      for g in range(bsz // LANES):
        sl = pl.ds(g * LANES, LANES)
        hs = plsc.load_gather(h2v, [sidv[sl]])
        hd = plsc.load_gather(h2v, [didv[sl]])
        al = _lrelu(cs * hs + cd * hd)
        cb = _lrelu(s2max + cd * hd)
        ev = jnp.exp(al - cb)
        val = ev * hs
        for r in range(LANES):
          vs = take(val, r)
          es = take(ev, r)
          row = jnp.where(lane == 0, vs, jnp.where(lane == 1, es, zv))
          stage[g * LANES + r, :] = row
      pltpu.sync_copy(stage, acc.at[didv], add=True)

    plsc.subcore_barrier()
    pltpu.sync_copy(acc.at[pl.ds(s * srows, srows)],
                    out_h.at[c, pl.ds(s * srows, srows)])

  return k(src, dst, h2flat, params, z16)


def _tce(out2, b2c, *, npad, rblk, interpret):
  nblk = npad // rblk

  def body(o_ref, b_ref, out_ref):
    o = o_ref[0] + o_ref[1]
    out_ref[...] = o[:, 0:1] / (o[:, 1:2] + 1e-16) + b_ref[...]

  return pl.pallas_call(
      body,
      grid=(nblk,),
      in_specs=[
          pl.BlockSpec((NC, rblk, LANES), lambda i: (0, i, 0)),
          pl.BlockSpec((1, 1), lambda i: (0, 0)),
      ],
      out_specs=pl.BlockSpec((rblk, 1), lambda i: (i, 0)),
      out_shape=_sds((npad, 1), F32),
      interpret=interpret,
  )(out2, b2c)


def _pipeline(x, edge_index, W1, att_src1, att_dst1, b1, W2, att_src2,
              att_dst2, b2, *, npad, epad, rblk, bsz, interpret=False):
  n, in_ch = x.shape
  heads, hid = att_src1.shape
  hidtot = heads * hid

  ei = edge_index.astype(I32)
  loop = jnp.arange(n, dtype=I32)
  e1 = ei.shape[1] + n
  src = jnp.concatenate(
      [ei[0], loop, jnp.zeros((epad - e1,), I32)])
  dst = jnp.concatenate(
      [ei[1], loop, jnp.full((epad - e1,), n, I32)])
  xp = jnp.pad(x, ((0, npad - n), (0, 0)))
  Asrc = jnp.pad((jnp.eye(heads, dtype=F32)[:, None, :]
                  * att_src1[:, :, None]).reshape(hidtot, heads),
                 ((0, 0), (0, LANES - heads)))
  Adst = jnp.pad((jnp.eye(heads, dtype=F32)[:, None, :]
                  * att_dst1[:, :, None]).reshape(hidtot, heads),
                 ((0, 0), (0, LANES - heads)))
  z16 = jnp.zeros((npad, LANES), F32)
  z32 = jnp.zeros((npad, 32), F32)

  h, asrc, adst, smax = _tc1(
      xp, W1, Asrc, Adst, npad=npad, rblk=rblk, heads=heads,
      interpret=interpret)
  ext, denp = _sca(
      src, dst, asrc, adst, smax.reshape(-1), z16, npad=npad, epad=epad,
      bsz=bsz, heads=heads, interpret=interpret)
  h2d = h.reshape(npad * 16, 32)
  idx16 = src[None, :] * 16 + jnp.arange(16, dtype=I32)[:, None]
  U = _scb(idx16, dst, ext, h2d, z32, npad=npad, epad=epad, bsz=bsz,
           interpret=interpret)
  b1r = b1.reshape(16, 32)
  W2r = W2[:, 0].reshape(16, 32)
  h2col, mm = _tcc(U, denp, b1r, W2r, n=n, npad=npad, rblk=rblk, heads=heads,
                   interpret=interpret)
  params = jnp.concatenate(
      [att_src2.reshape(-1)[:1], att_dst2.reshape(-1)[:1], mm[0],
       jnp.zeros((LANES - 4,), F32)])
  out2 = _scd(src, dst, h2col.reshape(npad), params, z16, npad=npad,
              epad=epad, bsz=bsz, interpret=interpret)
  outp = _tce(out2, b2.reshape(1, 1), npad=npad, rblk=rblk,
              interpret=interpret)
  return outp[:n]


def kernel(x, edge_index, W1, att_src1, att_dst1, b1, W2, att_src2, att_dst2,
           b2):
  return _pipeline(
      x, edge_index, W1, att_src1, att_dst1, b1, W2, att_src2, att_dst2, b2,
      npad=50176, epad=851968, rblk=1024, bsz=128)


# speedup vs baseline: 28.7899x; 2.0287x over previous
import functools

import jax
import jax.numpy as jnp
from jax import lax
from jax.experimental import pallas as pl
from jax.experimental.pallas import tpu as pltpu
from jax.experimental.pallas import tpu_sc as plsc

NC = 2
NS = 16
LANES = 16

F32 = jnp.float32
I32 = jnp.int32


def _lrelu(v):
  return jnp.where(v >= 0, v, 0.2 * v)


def _sds(shape, dtype):
  return jax.ShapeDtypeStruct(shape, dtype)


def _tc1(xp, W1, Asrc, Adst, *, npad, rblk, heads, interpret):
  nblk = npad // rblk
  in_ch = xp.shape[1]
  hidtot = W1.shape[1]

  def body(x_ref, w_ref, as_ref, ad_ref, h_ref, asrc_ref, adst_ref, smax_ref):
    h = jnp.dot(x_ref[...], w_ref[...], preferred_element_type=F32)
    h_ref[...] = h
    a_s = jnp.dot(h, as_ref[...], preferred_element_type=F32)
    a_d = jnp.dot(h, ad_ref[...], preferred_element_type=F32)
    asrc_ref[...] = a_s
    adst_ref[...] = a_d
    bm = jnp.max(a_s, axis=0, keepdims=True)

    @pl.when(pl.program_id(0) == 0)
    def _():
      smax_ref[...] = bm

    @pl.when(pl.program_id(0) > 0)
    def _():
      smax_ref[...] = jnp.maximum(smax_ref[...], bm)

  return pl.pallas_call(
      body,
      grid=(nblk,),
      in_specs=[
          pl.BlockSpec((rblk, in_ch), lambda i: (i, 0)),
          pl.BlockSpec((in_ch, hidtot), lambda i: (0, 0)),
          pl.BlockSpec((hidtot, LANES), lambda i: (0, 0)),
          pl.BlockSpec((hidtot, LANES), lambda i: (0, 0)),
      ],
      out_specs=[
          pl.BlockSpec((rblk, hidtot), lambda i: (i, 0)),
          pl.BlockSpec((rblk, LANES), lambda i: (i, 0)),
          pl.BlockSpec((rblk, LANES), lambda i: (i, 0)),
          pl.BlockSpec((1, LANES), lambda i: (0, 0)),
      ],
      out_shape=[
          _sds((npad, hidtot), F32),
          _sds((npad, LANES), F32),
          _sds((npad, LANES), F32),
          _sds((1, LANES), F32),
      ],
      interpret=interpret,
  )(xp, W1, Asrc, Adst)


def _sca(src, dst, asrc, adst, smax, z16, *, npad, epad, bsz, heads,
         interpret):
  nw = NC * NS
  chunk = epad // nw
  nbatch = chunk // bsz
  srows = npad // NS
  mesh = plsc.VectorSubcoreMesh(
      core_axis_name="c", subcore_axis_name="s", num_cores=NC, num_subcores=NS)

  @functools.partial(
      pl.kernel,
      out_type=(_sds((heads, epad), F32), _sds((NC, npad, LANES), F32)),
      mesh=mesh,
      compiler_params=pltpu.CompilerParams(needs_layout_passes=False, use_tc_tiling_on_sc=False),
      scratch_types=[
          pltpu.VMEM((bsz,), I32),
          pltpu.VMEM((bsz,), I32),
          pltpu.VMEM((bsz, LANES), F32),
          pltpu.VMEM((bsz, LANES), F32),
          pltpu.VMEM((bsz, LANES), F32),
          pltpu.VMEM((bsz * LANES,), F32),
          pltpu.VMEM((heads, bsz), F32),
          pltpu.VMEM((LANES,), F32),
          pltpu.VMEM_SHARED((npad, LANES), F32),
          pltpu.SemaphoreType.DMA,
          pltpu.SemaphoreType.DMA,
      ],
      interpret=interpret,
  )
  def k(src_h, dst_h, asrc_h, adst_h, smax_h, z16_h, ext_h, den_h,
        sidv, didv, sbuf, dbuf, aos, aosf, soa, smv, acc, sem1, sem2):
    c = lax.axis_index("c")
    s = lax.axis_index("s")
    w = s * NC + c
    pltpu.sync_copy(smax_h, smv)
    pltpu.sync_copy(z16_h.at[pl.ds(s * srows, srows)],
                    acc.at[pl.ds(s * srows, srows)])
    plsc.subcore_barrier()

    @pl.loop(0, nbatch)
    def _(t):
      off = w * chunk + t * bsz
      pltpu.sync_copy(src_h.at[pl.ds(off, bsz)], sidv)
      pltpu.sync_copy(dst_h.at[pl.ds(off, bsz)], didv)
      d1 = pltpu.async_copy(asrc_h.at[sidv], sbuf, sem1)
      d2 = pltpu.async_copy(adst_h.at[didv], dbuf, sem2)
      d1.wait()
      d2.wait()
      smaxv = smv[...]
      for r in range(bsz):
        sv = sbuf[r, :]
        dv = dbuf[r, :]
        al = _lrelu(sv + dv)
        cb = _lrelu(smaxv + dv)
        ev = jnp.exp(al - cb)
        aos[r, :] = ev
        aosf[pl.ds(r * LANES, LANES)] = ev
      lane = jax.lax.iota(I32, LANES)
      for g in range(bsz // LANES):
        ridx = (g * LANES + lane) * LANES
        for kk in range(heads):
          col = plsc.load_gather(aosf, [ridx + kk])
          soa[kk, pl.ds(g * LANES, LANES)] = col
      pltpu.sync_copy(soa, ext_h.at[:, pl.ds(off, bsz)])
      pltpu.sync_copy(aos, acc.at[didv], add=True)

    plsc.subcore_barrier()
    pltpu.sync_copy(acc.at[pl.ds(s * srows, srows)],
                    den_h.at[c, pl.ds(s * srows, srows)])

  return k(src, dst, asrc, adst, smax, z16)


def _scb(idx16, dst, ext, h2d, z32, *, npad, epad, bsz, interpret):
  chunk = epad // NS
  nbatch = chunk // bsz
  srows = npad // NS
  npass = 8
  mesh = plsc.VectorSubcoreMesh(
      core_axis_name="c", subcore_axis_name="s", num_cores=NC, num_subcores=NS)

  @functools.partial(
      pl.kernel,
      out_type=_sds((2 * npass, npad, 32), F32),
      mesh=mesh,
      compiler_params=pltpu.CompilerParams(needs_layout_passes=False, use_tc_tiling_on_sc=False),
      scratch_types=[
          pltpu.VMEM((bsz,), I32), pltpu.VMEM((bsz,), I32),
          pltpu.VMEM((bsz,), I32), pltpu.VMEM((bsz,), I32),
          pltpu.VMEM((bsz,), F32), pltpu.VMEM((bsz,), F32),
          pltpu.VMEM((bsz, 32), F32), pltpu.VMEM((bsz, 32), F32),
          pltpu.VMEM_SHARED((npad, 32), F32),
          pltpu.SemaphoreType.DMA, pltpu.SemaphoreType.DMA,
          pltpu.SemaphoreType.DMA, pltpu.SemaphoreType.DMA,
          pltpu.SemaphoreType.DMA, pltpu.SemaphoreType.DMA,
      ],
      interpret=interpret,
  )
  def k(idx16_h, dst_h, ext_h, h2d_h, z32_h, u_h,
        didv0, didv1, idxv0, idxv1, exv0, exv1, hbuf0, hbuf1, acc,
        asem0, asem1, gsem0, gsem1, ssem0, ssem1):
    c = lax.axis_index("c")
    s = lax.axis_index("s")
    didv = (didv0, didv1)
    idxv = (idxv0, idxv1)
    exv = (exv0, exv1)
    hbuf = (hbuf0, hbuf1)
    asem = (asem0, asem1)
    gsem = (gsem0, gsem1)
    ssem = (ssem0, ssem1)

    @pl.loop(0, npass)
    def _(p):
      j = c * npass + p
      head = j // 2
      pltpu.sync_copy(z32_h.at[pl.ds(s * srows, srows)],
                      acc.at[pl.ds(s * srows, srows)])
      plsc.subcore_barrier()

      @pl.loop(0, nbatch // 2)
      def _(m):
        la = [None, None]
        for b in range(2):
          off = s * chunk + (m * 2 + b) * bsz
          la[b] = (
              pltpu.async_copy(idx16_h.at[j, pl.ds(off, bsz)], idxv[b],
                               asem[b]),
              pltpu.async_copy(dst_h.at[pl.ds(off, bsz)], didv[b], asem[b]),
              pltpu.async_copy(ext_h.at[head, pl.ds(off, bsz)], exv[b],
                               asem[b]),
          )
        gd = [None, None]
        for b in range(2):
          for d in la[b]:
            d.wait()
          gd[b] = pltpu.async_copy(h2d_h.at[idxv[b]], hbuf[b], gsem[b])
        sd = [None, None]
        for b in range(2):
          gd[b].wait()
          hb = hbuf[b]
          exb = exv[b]
          for g in range(bsz // LANES):
            evec = exb[pl.ds(g * LANES, LANES)]
            for rr in range(LANES):
              r = g * LANES + rr
              ev = evec.at[jnp.full((LANES,), rr, I32)].get(
                  mode="promise_in_bounds")
              hb[r, 0:16] = hb[r, 0:16] * ev
              hb[r, 16:32] = hb[r, 16:32] * ev
          sd[b] = pltpu.async_copy(hb, acc.at[didv[b]], ssem[b], add=True)
        for b in range(2):
          sd[b].wait()

      plsc.subcore_barrier()
      pltpu.sync_copy(acc.at[pl.ds(s * srows, srows)],
                      u_h.at[j, pl.ds(s * srows, srows)])
      plsc.subcore_barrier()

  return k(idx16, dst, ext, h2d, z32)


def _tcc(U, denp, b1r, W2r, *, n, npad, rblk, heads, interpret):
  nblk = npad // rblk

  def body(u_ref, dp_ref, b1_ref, w2_ref, h2_ref, mm_ref):
    den = dp_ref[0, :, 0:heads] + dp_ref[1, :, 0:heads] + 1e-16
    acc = jnp.zeros((rblk, 1), F32)
    for j in range(16):
      u = u_ref[j]
      dj = den[:, j // 2][:, None]
      hj = u / dj + b1_ref[j][None, :]
      hj = jnp.where(hj > 0, hj, jnp.exp(hj) - 1.0)
      acc = acc + jnp.dot(hj, w2_ref[j][:, None], preferred_element_type=F32)
    h2_ref[...] = acc
    rows = pl.program_id(0) * rblk + lax.broadcasted_iota(I32, (rblk, 1), 0)
    valid = rows < n
    hx = jnp.max(jnp.where(valid, acc, -jnp.inf)).reshape(1, 1)
    hn = jnp.min(jnp.where(valid, acc, jnp.inf)).reshape(1, 1)
    bm = jnp.concatenate([hn, hx], axis=1)

    @pl.when(pl.program_id(0) == 0)
    def _():
      mm_ref[...] = bm

    @pl.when(pl.program_id(0) > 0)
    def _():
      prev = mm_ref[...]
      mm_ref[...] = jnp.concatenate(
          [jnp.minimum(prev[:, 0:1], bm[:, 0:1]),
           jnp.maximum(prev[:, 1:2], bm[:, 1:2])], axis=1)

  return pl.pallas_call(
      body,
      grid=(nblk,),
      in_specs=[
          pl.BlockSpec((16, rblk, 32), lambda i: (0, i, 0)),
          pl.BlockSpec((NC, rblk, LANES), lambda i: (0, i, 0)),
          pl.BlockSpec((16, 32), lambda i: (0, 0)),
          pl.BlockSpec((16, 32), lambda i: (0, 0)),
      ],
      out_specs=[
          pl.BlockSpec((rblk, 1), lambda i: (i, 0)),
          pl.BlockSpec((1, 2), lambda i: (0, 0)),
      ],
      out_shape=[_sds((npad, 1), F32), _sds((1, 2), F32)],
      interpret=interpret,
  )(U, denp, b1r, W2r)


def _scd(src, dst, h2flat, params, z16, *, npad, epad, bsz, interpret):
  nw = NC * NS
  chunk = epad // nw
  nbatch = chunk // bsz
  srows = npad // NS
  mesh = plsc.VectorSubcoreMesh(
      core_axis_name="c", subcore_axis_name="s", num_cores=NC, num_subcores=NS)

  @functools.partial(
      pl.kernel,
      out_type=_sds((NC, npad, LANES), F32),
      mesh=mesh,
      compiler_params=pltpu.CompilerParams(needs_layout_passes=False, use_tc_tiling_on_sc=False),
      scratch_types=[
          pltpu.VMEM((npad,), F32),
          pltpu.VMEM((bsz,), I32),
          pltpu.VMEM((bsz,), I32),
          pltpu.VMEM((bsz, LANES), F32),
          pltpu.VMEM((LANES,), F32),
          pltpu.VMEM_SHARED((npad, LANES), F32),
      ],
      interpret=interpret,
  )
  def k(src_h, dst_h, h2_h, par_h, z16_h, out_h,
        h2v, sidv, didv, stage, pv, acc):
    c = lax.axis_index("c")
    s = lax.axis_index("s")
    w = s * NC + c
    pltpu.sync_copy(h2_h, h2v)
    pltpu.sync_copy(par_h, pv)
    pltpu.sync_copy(z16_h.at[pl.ds(s * srows, srows)],
                    acc.at[pl.ds(s * srows, srows)])
    plsc.subcore_barrier()
    lane = jax.lax.iota(I32, LANES)
    pvv = pv[...]
    take = lambda v, i: v.at[jnp.full((LANES,), i, I32)].get(
        mode="promise_in_bounds")
    cs = take(pvv, 0)
    cd = take(pvv, 1)
    mnv = take(pvv, 2)
    mxv = take(pvv, 3)
    s2max = jnp.maximum(cs * mxv, cs * mnv)
    zv = jnp.zeros((LANES,), F32)

    @pl.loop(0, nbatch)
    def _(t):
      off = w * chunk + t * bsz
      pltpu.sync_copy(src_h.at[pl.ds(off, bsz)], sidv)
      pltpu.sync_copy(dst_h.at[pl.ds(off, bsz)], didv)
      for g in range(bsz // LANES):
        sl = pl.ds(g * LANES, LANES)
        hs = plsc.load_gather(h2v, [sidv[sl]])
        hd = plsc.load_gather(h2v, [didv[sl]])
        al = _lrelu(cs * hs + cd * hd)
        cb = _lrelu(s2max + cd * hd)
        ev = jnp.exp(al - cb)
        val = ev * hs
        for r in range(LANES):
          vs = take(val, r)
          es = take(ev, r)
          row = jnp.where(lane == 0, vs, jnp.where(lane == 1, es, zv))
          stage[g * LANES + r, :] = row
      pltpu.sync_copy(stage, acc.at[didv], add=True)

    plsc.subcore_barrier()
    pltpu.sync_copy(acc.at[pl.ds(s * srows, srows)],
                    out_h.at[c, pl.ds(s * srows, srows)])

  return k(src, dst, h2flat, params, z16)


def _tce(out2, b2c, *, npad, rblk, interpret):
  nblk = npad // rblk

  def body(o_ref, b_ref, out_ref):
    o = o_ref[0] + o_ref[1]
    out_ref[...] = o[:, 0:1] / (o[:, 1:2] + 1e-16) + b_ref[...]

  return pl.pallas_call(
      body,
      grid=(nblk,),
      in_specs=[
          pl.BlockSpec((NC, rblk, LANES), lambda i: (0, i, 0)),
          pl.BlockSpec((1, 1), lambda i: (0, 0)),
      ],
      out_specs=pl.BlockSpec((rblk, 1), lambda i: (i, 0)),
      out_shape=_sds((npad, 1), F32),
      interpret=interpret,
  )(out2, b2c)


def _pipeline(x, edge_index, W1, att_src1, att_dst1, b1, W2, att_src2,
              att_dst2, b2, *, npad, epad, rblk, bsz, interpret=False):
  n, in_ch = x.shape
  heads, hid = att_src1.shape
  hidtot = heads * hid

  ei = edge_index.astype(I32)
  loop = jnp.arange(n, dtype=I32)
  e1 = ei.shape[1] + n
  src = jnp.concatenate(
      [ei[0], loop, jnp.zeros((epad - e1,), I32)])
  dst = jnp.concatenate(
      [ei[1], loop, jnp.full((epad - e1,), n, I32)])
  xp = jnp.pad(x, ((0, npad - n), (0, 0)))
  Asrc = jnp.pad((jnp.eye(heads, dtype=F32)[:, None, :]
                  * att_src1[:, :, None]).reshape(hidtot, heads),
                 ((0, 0), (0, LANES - heads)))
  Adst = jnp.pad((jnp.eye(heads, dtype=F32)[:, None, :]
                  * att_dst1[:, :, None]).reshape(hidtot, heads),
                 ((0, 0), (0, LANES - heads)))
  z16 = jnp.zeros((npad, LANES), F32)
  z32 = jnp.zeros((npad, 32), F32)

  h, asrc, adst, smax = _tc1(
      xp, W1, Asrc, Adst, npad=npad, rblk=rblk, heads=heads,
      interpret=interpret)
  ext, denp = _sca(
      src, dst, asrc, adst, smax.reshape(-1), z16, npad=npad, epad=epad,
      bsz=bsz, heads=heads, interpret=interpret)
  h2d = h.reshape(npad * 16, 32)
  idx16 = src[None, :] * 16 + jnp.arange(16, dtype=I32)[:, None]
  U = _scb(idx16, dst, ext, h2d, z32, npad=npad, epad=epad, bsz=bsz,
           interpret=interpret)
  b1r = b1.reshape(16, 32)
  W2r = W2[:, 0].reshape(16, 32)
  h2col, mm = _tcc(U, denp, b1r, W2r, n=n, npad=npad, rblk=rblk, heads=heads,
                   interpret=interpret)
  params = jnp.concatenate(
      [att_src2.reshape(-1)[:1], att_dst2.reshape(-1)[:1], mm[0],
       jnp.zeros((LANES - 4,), F32)])
  out2 = _scd(src, dst, h2col.reshape(npad), params, z16, npad=npad,
              epad=epad, bsz=bsz, interpret=interpret)
  outp = _tce(out2, b2.reshape(1, 1), npad=npad, rblk=rblk,
              interpret=interpret)
  return outp[:n]


def kernel(x, edge_index, W1, att_src1, att_dst1, b1, W2, att_src2, att_dst2,
           b2):
  return _pipeline(
      x, edge_index, W1, att_src1, att_dst1, b1, W2, att_src2, att_dst2, b2,
      npad=50176, epad=851968, rblk=1024, bsz=128)


# speedup vs baseline: 30.6676x; 1.0652x over previous
import functools

import jax
import jax.numpy as jnp
from jax import lax
from jax.experimental import pallas as pl
from jax.experimental.pallas import tpu as pltpu
from jax.experimental.pallas import tpu_sc as plsc

NC = 2
NS = 16
LANES = 16

F32 = jnp.float32
I32 = jnp.int32


def _lrelu(v):
  return jnp.where(v >= 0, v, 0.2 * v)


def _sds(shape, dtype):
  return jax.ShapeDtypeStruct(shape, dtype)


def _tc1(xp, W1, Asrc, Adst, *, npad, rblk, heads, interpret):
  nblk = npad // rblk
  in_ch = xp.shape[1]
  hidtot = W1.shape[1]

  def body(x_ref, w_ref, as_ref, ad_ref, h_ref, asrc_ref, adst_ref, smax_ref):
    h = jnp.dot(x_ref[...], w_ref[...], preferred_element_type=F32)
    h_ref[...] = h
    a_s = jnp.dot(h, as_ref[...], preferred_element_type=F32)
    a_d = jnp.dot(h, ad_ref[...], preferred_element_type=F32)
    asrc_ref[...] = a_s
    adst_ref[...] = a_d
    bm = jnp.max(a_s, axis=0, keepdims=True)

    @pl.when(pl.program_id(0) == 0)
    def _():
      smax_ref[...] = bm

    @pl.when(pl.program_id(0) > 0)
    def _():
      smax_ref[...] = jnp.maximum(smax_ref[...], bm)

  return pl.pallas_call(
      body,
      grid=(nblk,),
      in_specs=[
          pl.BlockSpec((rblk, in_ch), lambda i: (i, 0)),
          pl.BlockSpec((in_ch, hidtot), lambda i: (0, 0)),
          pl.BlockSpec((hidtot, LANES), lambda i: (0, 0)),
          pl.BlockSpec((hidtot, LANES), lambda i: (0, 0)),
      ],
      out_specs=[
          pl.BlockSpec((rblk, hidtot), lambda i: (i, 0)),
          pl.BlockSpec((rblk, LANES), lambda i: (i, 0)),
          pl.BlockSpec((rblk, LANES), lambda i: (i, 0)),
          pl.BlockSpec((1, LANES), lambda i: (0, 0)),
      ],
      out_shape=[
          _sds((npad, hidtot), F32),
          _sds((npad, LANES), F32),
          _sds((npad, LANES), F32),
          _sds((1, LANES), F32),
      ],
      interpret=interpret,
  )(xp, W1, Asrc, Adst)


def _sca(src, dst, asrc, adst, smax, z16, *, npad, epad, bsz, heads,
         interpret):
  nw = NC * NS
  chunk = epad // nw
  nbatch = chunk // bsz
  srows = npad // NS
  mesh = plsc.VectorSubcoreMesh(
      core_axis_name="c", subcore_axis_name="s", num_cores=NC, num_subcores=NS)

  @functools.partial(
      pl.kernel,
      out_type=(_sds((heads, epad), F32), _sds((NC, npad, LANES), F32)),
      mesh=mesh,
      compiler_params=pltpu.CompilerParams(needs_layout_passes=False, use_tc_tiling_on_sc=False),
      scratch_types=[
          pltpu.VMEM((bsz,), I32),
          pltpu.VMEM((bsz,), I32),
          pltpu.VMEM((bsz, LANES), F32),
          pltpu.VMEM((bsz, LANES), F32),
          pltpu.VMEM((bsz, LANES), F32),
          pltpu.VMEM((bsz * LANES,), F32),
          pltpu.VMEM((heads, bsz), F32),
          pltpu.VMEM((LANES,), F32),
          pltpu.VMEM_SHARED((npad, LANES), F32),
          pltpu.SemaphoreType.DMA,
          pltpu.SemaphoreType.DMA,
      ],
      interpret=interpret,
  )
  def k(src_h, dst_h, asrc_h, adst_h, smax_h, z16_h, ext_h, den_h,
        sidv, didv, sbuf, dbuf, aos, aosf, soa, smv, acc, sem1, sem2):
    c = lax.axis_index("c")
    s = lax.axis_index("s")
    w = s * NC + c
    pltpu.sync_copy(smax_h, smv)
    pltpu.sync_copy(z16_h.at[pl.ds(s * srows, srows)],
                    acc.at[pl.ds(s * srows, srows)])
    plsc.subcore_barrier()

    @pl.loop(0, nbatch)
    def _(t):
      off = w * chunk + t * bsz
      pltpu.sync_copy(src_h.at[pl.ds(off, bsz)], sidv)
      pltpu.sync_copy(dst_h.at[pl.ds(off, bsz)], didv)
      d1 = pltpu.async_copy(asrc_h.at[sidv], sbuf, sem1)
      d2 = pltpu.async_copy(adst_h.at[didv], dbuf, sem2)
      d1.wait()
      d2.wait()
      smaxv = smv[...]
      for r in range(bsz):
        sv = sbuf[r, :]
        dv = dbuf[r, :]
        al = _lrelu(sv + dv)
        cb = _lrelu(smaxv + dv)
        ev = jnp.exp(al - cb)
        aos[r, :] = ev
        aosf[pl.ds(r * LANES, LANES)] = ev
      lane = jax.lax.iota(I32, LANES)
      for g in range(bsz // LANES):
        ridx = (g * LANES + lane) * LANES
        for kk in range(heads):
          col = plsc.load_gather(aosf, [ridx + kk])
          soa[kk, pl.ds(g * LANES, LANES)] = col
      pltpu.sync_copy(soa, ext_h.at[:, pl.ds(off, bsz)])
      pltpu.sync_copy(aos, acc.at[didv], add=True)

    plsc.subcore_barrier()
    pltpu.sync_copy(acc.at[pl.ds(s * srows, srows)],
                    den_h.at[c, pl.ds(s * srows, srows)])

  return k(src, dst, asrc, adst, smax, z16)


def _scb(idx16, dst, ext, h2d, z32, *, npad, epad, bsz, interpret):
  chunk = epad // NS
  nbatch = chunk // bsz
  srows = npad // NS
  npass = 8
  mesh = plsc.VectorSubcoreMesh(
      core_axis_name="c", subcore_axis_name="s", num_cores=NC, num_subcores=NS)

  @functools.partial(
      pl.kernel,
      out_type=_sds((2 * npass, npad, 32), F32),
      mesh=mesh,
      compiler_params=pltpu.CompilerParams(needs_layout_passes=False, use_tc_tiling_on_sc=False),
      scratch_types=[
          pltpu.VMEM((bsz,), I32), pltpu.VMEM((bsz,), I32),
          pltpu.VMEM((bsz,), I32), pltpu.VMEM((bsz,), I32),
          pltpu.VMEM((bsz,), F32), pltpu.VMEM((bsz,), F32),
          pltpu.VMEM((bsz, 32), F32), pltpu.VMEM((bsz, 32), F32),
          pltpu.VMEM_SHARED((npad, 32), F32),
          pltpu.SemaphoreType.DMA, pltpu.SemaphoreType.DMA,
          pltpu.SemaphoreType.DMA, pltpu.SemaphoreType.DMA,
          pltpu.SemaphoreType.DMA, pltpu.SemaphoreType.DMA,
      ],
      interpret=interpret,
  )
  def k(idx16_h, dst_h, ext_h, h2d_h, z32_h, u_h,
        didv0, didv1, idxv0, idxv1, exv0, exv1, hbuf0, hbuf1, acc,
        asem0, asem1, gsem0, gsem1, ssem0, ssem1):
    c = lax.axis_index("c")
    s = lax.axis_index("s")
    didv = (didv0, didv1)
    idxv = (idxv0, idxv1)
    exv = (exv0, exv1)
    hbuf = (hbuf0, hbuf1)
    asem = (asem0, asem1)
    gsem = (gsem0, gsem1)
    ssem = (ssem0, ssem1)

    @pl.loop(0, npass)
    def _(p):
      j = c * npass + p
      head = j // 2
      pltpu.sync_copy(z32_h.at[pl.ds(s * srows, srows)],
                      acc.at[pl.ds(s * srows, srows)])
      plsc.subcore_barrier()

      for b in range(2):
        off = s * chunk + b * bsz
        pltpu.async_copy(idx16_h.at[j, pl.ds(off, bsz)], idxv[b], asem[b])
        pltpu.async_copy(ext_h.at[head, pl.ds(off, bsz)], exv[b], asem[b])
        pltpu.async_copy(dst_h.at[pl.ds(off, bsz)], didv[b], asem[b])

      @pl.loop(0, nbatch // 2)
      def _(m):
        gd = [None, None]
        for b in range(2):
          pltpu.make_async_copy(idx16_h.at[j, pl.ds(0, bsz)], idxv[b],
                                asem[b]).wait()
          pltpu.make_async_copy(ext_h.at[head, pl.ds(0, bsz)], exv[b],
                                asem[b]).wait()
          pltpu.make_async_copy(dst_h.at[pl.ds(0, bsz)], didv[b],
                                asem[b]).wait()
          gd[b] = pltpu.async_copy(h2d_h.at[idxv[b]], hbuf[b], gsem[b])
        more = m < nbatch // 2 - 1
        noff = s * chunk + (m + 1) * 2 * bsz
        sd = [None, None]
        for b in range(2):
          gd[b].wait()

          @pl.when(more)
          def _(b=b):
            pltpu.async_copy(idx16_h.at[j, pl.ds(noff + b * bsz, bsz)],
                             idxv[b], asem[b])

          hb = hbuf[b]
          exb = exv[b]
          for g in range(bsz // LANES):
            evec = exb[pl.ds(g * LANES, LANES)]
            for rr in range(LANES):
              r = g * LANES + rr
              ev = evec.at[jnp.full((LANES,), rr, I32)].get(
                  mode="promise_in_bounds")
              hb[r, 0:16] = hb[r, 0:16] * ev
              hb[r, 16:32] = hb[r, 16:32] * ev

          @pl.when(more)
          def _(b=b):
            pltpu.async_copy(ext_h.at[head, pl.ds(noff + b * bsz, bsz)],
                             exv[b], asem[b])

          sd[b] = pltpu.async_copy(hb, acc.at[didv[b]], ssem[b], add=True)
        for b in range(2):
          sd[b].wait()

          @pl.when(more)
          def _(b=b):
            pltpu.async_copy(dst_h.at[pl.ds(noff + b * bsz, bsz)], didv[b],
                             asem[b])

      plsc.subcore_barrier()
      pltpu.sync_copy(acc.at[pl.ds(s * srows, srows)],
                      u_h.at[j, pl.ds(s * srows, srows)])
      plsc.subcore_barrier()

  return k(idx16, dst, ext, h2d, z32)


def _tcc(U, denp, b1r, W2r, *, n, npad, rblk, heads, interpret):
  nblk = npad // rblk

  def body(u_ref, dp_ref, b1_ref, w2_ref, h2_ref, mm_ref):
    den = dp_ref[0, :, 0:heads] + dp_ref[1, :, 0:heads] + 1e-16
    acc = jnp.zeros((rblk, 1), F32)
    for j in range(16):
      u = u_ref[j]
      dj = den[:, j // 2][:, None]
      hj = u / dj + b1_ref[j][None, :]
      hj = jnp.where(hj > 0, hj, jnp.exp(hj) - 1.0)
      acc = acc + jnp.dot(hj, w2_ref[j][:, None], preferred_element_type=F32)
    h2_ref[...] = acc
    rows = pl.program_id(0) * rblk + lax.broadcasted_iota(I32, (rblk, 1), 0)
    valid = rows < n
    hx = jnp.max(jnp.where(valid, acc, -jnp.inf)).reshape(1, 1)
    hn = jnp.min(jnp.where(valid, acc, jnp.inf)).reshape(1, 1)
    bm = jnp.concatenate([hn, hx], axis=1)

    @pl.when(pl.program_id(0) == 0)
    def _():
      mm_ref[...] = bm

    @pl.when(pl.program_id(0) > 0)
    def _():
      prev = mm_ref[...]
      mm_ref[...] = jnp.concatenate(
          [jnp.minimum(prev[:, 0:1], bm[:, 0:1]),
           jnp.maximum(prev[:, 1:2], bm[:, 1:2])], axis=1)

  return pl.pallas_call(
      body,
      grid=(nblk,),
      in_specs=[
          pl.BlockSpec((16, rblk, 32), lambda i: (0, i, 0)),
          pl.BlockSpec((NC, rblk, LANES), lambda i: (0, i, 0)),
          pl.BlockSpec((16, 32), lambda i: (0, 0)),
          pl.BlockSpec((16, 32), lambda i: (0, 0)),
      ],
      out_specs=[
          pl.BlockSpec((rblk, 1), lambda i: (i, 0)),
          pl.BlockSpec((1, 2), lambda i: (0, 0)),
      ],
      out_shape=[_sds((npad, 1), F32), _sds((1, 2), F32)],
      interpret=interpret,
  )(U, denp, b1r, W2r)


def _scd(src, dst, h2flat, params, z16, *, npad, epad, bsz, interpret):
  nw = NC * NS
  chunk = epad // nw
  nbatch = chunk // bsz
  srows = npad // NS
  mesh = plsc.VectorSubcoreMesh(
      core_axis_name="c", subcore_axis_name="s", num_cores=NC, num_subcores=NS)

  @functools.partial(
      pl.kernel,
      out_type=_sds((NC, npad, LANES), F32),
      mesh=mesh,
      compiler_params=pltpu.CompilerParams(needs_layout_passes=False, use_tc_tiling_on_sc=False),
      scratch_types=[
          pltpu.VMEM((npad,), F32),
          pltpu.VMEM((bsz,), I32),
          pltpu.VMEM((bsz,), I32),
          pltpu.VMEM((bsz, LANES), F32),
          pltpu.VMEM((LANES,), F32),
          pltpu.VMEM_SHARED((npad, LANES), F32),
      ],
      interpret=interpret,
  )
  def k(src_h, dst_h, h2_h, par_h, z16_h, out_h,
        h2v, sidv, didv, stage, pv, acc):
    c = lax.axis_index("c")
    s = lax.axis_index("s")
    w = s * NC + c
    pltpu.sync_copy(h2_h, h2v)
    pltpu.sync_copy(par_h, pv)
    pltpu.sync_copy(z16_h.at[pl.ds(s * srows, srows)],
                    acc.at[pl.ds(s * srows, srows)])
    plsc.subcore_barrier()
    lane = jax.lax.iota(I32, LANES)
    pvv = pv[...]
    take = lambda v, i: v.at[jnp.full((LANES,), i, I32)].get(
        mode="promise_in_bounds")
    cs = take(pvv, 0)
    cd = take(pvv, 1)
    mnv = take(pvv, 2)
    mxv = take(pvv, 3)
    s2max = jnp.maximum(cs * mxv, cs * mnv)
    zv = jnp.zeros((LANES,), F32)

    @pl.loop(0, nbatch)
    def _(t):
      off = w * chunk + t * bsz
      pltpu.sync_copy(src_h.at[pl.ds(off, bsz)], sidv)
      pltpu.sync_copy(dst_h.at[pl.ds(off, bsz)], didv)
      for g in range(bsz // LANES):
        sl = pl.ds(g * LANES, LANES)
        hs = plsc.load_gather(h2v, [sidv[sl]])
        hd = plsc.load_gather(h2v, [didv[sl]])
        al = _lrelu(cs * hs + cd * hd)
        cb = _lrelu(s2max + cd * hd)
        ev = jnp.exp(al - cb)
        val = ev * hs
        for r in range(LANES):
          vs = take(val, r)
          es = take(ev, r)
          row = jnp.where(lane == 0, vs, jnp.where(lane == 1, es, zv))
          stage[g * LANES + r, :] = row
      pltpu.sync_copy(stage, acc.at[didv], add=True)

    plsc.subcore_barrier()
    pltpu.sync_copy(acc.at[pl.ds(s * srows, srows)],
                    out_h.at[c, pl.ds(s * srows, srows)])

  return k(src, dst, h2flat, params, z16)


def _tce(out2, b2c, *, npad, rblk, interpret):
  nblk = npad // rblk

  def body(o_ref, b_ref, out_ref):
    o = o_ref[0] + o_ref[1]
    out_ref[...] = o[:, 0:1] / (o[:, 1:2] + 1e-16) + b_ref[...]

  return pl.pallas_call(
      body,
      grid=(nblk,),
      in_specs=[
          pl.BlockSpec((NC, rblk, LANES), lambda i: (0, i, 0)),
          pl.BlockSpec((1, 1), lambda i: (0, 0)),
      ],
      out_specs=pl.BlockSpec((rblk, 1), lambda i: (i, 0)),
      out_shape=_sds((npad, 1), F32),
      interpret=interpret,
  )(out2, b2c)


def _pipeline(x, edge_index, W1, att_src1, att_dst1, b1, W2, att_src2,
              att_dst2, b2, *, npad, epad, rblk, bsz, interpret=False):
  n, in_ch = x.shape
  heads, hid = att_src1.shape
  hidtot = heads * hid

  ei = edge_index.astype(I32)
  loop = jnp.arange(n, dtype=I32)
  e1 = ei.shape[1] + n
  src = jnp.concatenate(
      [ei[0], loop, jnp.zeros((epad - e1,), I32)])
  dst = jnp.concatenate(
      [ei[1], loop, jnp.full((epad - e1,), n, I32)])
  xp = jnp.pad(x, ((0, npad - n), (0, 0)))
  Asrc = jnp.pad((jnp.eye(heads, dtype=F32)[:, None, :]
                  * att_src1[:, :, None]).reshape(hidtot, heads),
                 ((0, 0), (0, LANES - heads)))
  Adst = jnp.pad((jnp.eye(heads, dtype=F32)[:, None, :]
                  * att_dst1[:, :, None]).reshape(hidtot, heads),
                 ((0, 0), (0, LANES - heads)))
  z16 = jnp.zeros((npad, LANES), F32)
  z32 = jnp.zeros((npad, 32), F32)

  h, asrc, adst, smax = _tc1(
      xp, W1, Asrc, Adst, npad=npad, rblk=rblk, heads=heads,
      interpret=interpret)
  ext, denp = _sca(
      src, dst, asrc, adst, smax.reshape(-1), z16, npad=npad, epad=epad,
      bsz=bsz, heads=heads, interpret=interpret)
  h2d = h.reshape(npad * 16, 32)
  idx16 = src[None, :] * 16 + jnp.arange(16, dtype=I32)[:, None]
  U = _scb(idx16, dst, ext, h2d, z32, npad=npad, epad=epad, bsz=bsz,
           interpret=interpret)
  b1r = b1.reshape(16, 32)
  W2r = W2[:, 0].reshape(16, 32)
  h2col, mm = _tcc(U, denp, b1r, W2r, n=n, npad=npad, rblk=rblk, heads=heads,
                   interpret=interpret)
  params = jnp.concatenate(
      [att_src2.reshape(-1)[:1], att_dst2.reshape(-1)[:1], mm[0],
       jnp.zeros((LANES - 4,), F32)])
  out2 = _scd(src, dst, h2col.reshape(npad), params, z16, npad=npad,
              epad=epad, bsz=bsz, interpret=interpret)
  outp = _tce(out2, b2.reshape(1, 1), npad=npad, rblk=rblk,
              interpret=interpret)
  return outp[:n]


def kernel(x, edge_index, W1, att_src1, att_dst1, b1, W2, att_src2, att_dst2,
           b2):
  return _pipeline(
      x, edge_index, W1, att_src1, att_dst1, b1, W2, att_src2, att_dst2, b2,
      npad=50176, epad=851968, rblk=1024, bsz=128)


# speedup vs baseline: 38.1475x; 1.2439x over previous
import functools

import jax
import jax.numpy as jnp
from jax import lax
from jax.experimental import pallas as pl
from jax.experimental.pallas import tpu as pltpu
from jax.experimental.pallas import tpu_sc as plsc

NC = 2
NS = 16
LANES = 16

F32 = jnp.float32
I32 = jnp.int32


def _lrelu(v):
  return jnp.where(v >= 0, v, 0.2 * v)


def _sds(shape, dtype):
  return jax.ShapeDtypeStruct(shape, dtype)


def _tc1(xp, W1, Asrc, Adst, *, npad, rblk, heads, interpret):
  nblk = npad // rblk
  in_ch = xp.shape[1]
  hidtot = W1.shape[1]

  def body(x_ref, w_ref, as_ref, ad_ref, h_ref, asrc_ref, adst_ref, smax_ref):
    h = jnp.dot(x_ref[...], w_ref[...], preferred_element_type=F32)
    h_ref[...] = h
    a_s = jnp.dot(h, as_ref[...], preferred_element_type=F32)
    a_d = jnp.dot(h, ad_ref[...], preferred_element_type=F32)
    asrc_ref[...] = a_s
    adst_ref[...] = a_d
    bm = jnp.max(a_s, axis=0, keepdims=True)

    @pl.when(pl.program_id(0) == 0)
    def _():
      smax_ref[...] = bm

    @pl.when(pl.program_id(0) > 0)
    def _():
      smax_ref[...] = jnp.maximum(smax_ref[...], bm)

  return pl.pallas_call(
      body,
      grid=(nblk,),
      in_specs=[
          pl.BlockSpec((rblk, in_ch), lambda i: (i, 0)),
          pl.BlockSpec((in_ch, hidtot), lambda i: (0, 0)),
          pl.BlockSpec((hidtot, LANES), lambda i: (0, 0)),
          pl.BlockSpec((hidtot, LANES), lambda i: (0, 0)),
      ],
      out_specs=[
          pl.BlockSpec((rblk, hidtot), lambda i: (i, 0)),
          pl.BlockSpec((rblk, LANES), lambda i: (i, 0)),
          pl.BlockSpec((rblk, LANES), lambda i: (i, 0)),
          pl.BlockSpec((1, LANES), lambda i: (0, 0)),
      ],
      out_shape=[
          _sds((npad, hidtot), F32),
          _sds((npad, LANES), F32),
          _sds((npad, LANES), F32),
          _sds((1, LANES), F32),
      ],
      interpret=interpret,
  )(xp, W1, Asrc, Adst)


def _sca(src, dst, asrc, adst, smax, z16, *, npad, epad, bsz, heads,
         interpret):
  nw = NC * NS
  chunk = epad // nw
  nbatch = chunk // bsz
  srows = npad // NS
  mesh = plsc.VectorSubcoreMesh(
      core_axis_name="c", subcore_axis_name="s", num_cores=NC, num_subcores=NS)

  @functools.partial(
      pl.kernel,
      out_type=(_sds((heads, epad), F32), _sds((NC, npad, LANES), F32)),
      mesh=mesh,
      compiler_params=pltpu.CompilerParams(needs_layout_passes=False, use_tc_tiling_on_sc=False),
      scratch_types=[
          pltpu.VMEM((bsz,), I32),
          pltpu.VMEM((bsz,), I32),
          pltpu.VMEM((bsz, LANES), F32),
          pltpu.VMEM((bsz, LANES), F32),
          pltpu.VMEM((bsz, LANES), F32),
          pltpu.VMEM((bsz * LANES,), F32),
          pltpu.VMEM((heads, bsz), F32),
          pltpu.VMEM((LANES,), F32),
          pltpu.VMEM_SHARED((npad, LANES), F32),
          pltpu.SemaphoreType.DMA,
          pltpu.SemaphoreType.DMA,
      ],
      interpret=interpret,
  )
  def k(src_h, dst_h, asrc_h, adst_h, smax_h, z16_h, ext_h, den_h,
        sidv, didv, sbuf, dbuf, aos, aosf, soa, smv, acc, sem1, sem2):
    c = lax.axis_index("c")
    s = lax.axis_index("s")
    w = s * NC + c
    pltpu.sync_copy(smax_h, smv)
    pltpu.sync_copy(z16_h.at[pl.ds(s * srows, srows)],
                    acc.at[pl.ds(s * srows, srows)])
    plsc.subcore_barrier()

    @pl.loop(0, nbatch)
    def _(t):
      off = w * chunk + t * bsz
      pltpu.sync_copy(src_h.at[pl.ds(off, bsz)], sidv)
      pltpu.sync_copy(dst_h.at[pl.ds(off, bsz)], didv)
      d1 = pltpu.async_copy(asrc_h.at[sidv], sbuf, sem1)
      d2 = pltpu.async_copy(adst_h.at[didv], dbuf, sem2)
      d1.wait()
      d2.wait()
      smaxv = smv[...]
      for r in range(bsz):
        sv = sbuf[r, :]
        dv = dbuf[r, :]
        al = _lrelu(sv + dv)
        cb = _lrelu(smaxv + dv)
        ev = jnp.exp(al - cb)
        aos[r, :] = ev
        aosf[pl.ds(r * LANES, LANES)] = ev
      lane = jax.lax.iota(I32, LANES)
      for g in range(bsz // LANES):
        ridx = (g * LANES + lane) * LANES
        for kk in range(heads):
          col = plsc.load_gather(aosf, [ridx + kk])
          soa[kk, pl.ds(g * LANES, LANES)] = col
      pltpu.sync_copy(soa, ext_h.at[:, pl.ds(off, bsz)])
      pltpu.sync_copy(aos, acc.at[didv], add=True)

    plsc.subcore_barrier()
    pltpu.sync_copy(acc.at[pl.ds(s * srows, srows)],
                    den_h.at[c, pl.ds(s * srows, srows)])

  return k(src, dst, asrc, adst, smax, z16)


def _scb(idx16, dst, ext, h2d, z32, *, npad, epad, bsz, interpret):
  chunk = epad // NS
  nbatch = chunk // bsz
  srows = npad // NS
  npass = 8
  mesh = plsc.VectorSubcoreMesh(
      core_axis_name="c", subcore_axis_name="s", num_cores=NC, num_subcores=NS)

  @functools.partial(
      pl.kernel,
      out_type=_sds((2 * npass, npad, 32), F32),
      mesh=mesh,
      compiler_params=pltpu.CompilerParams(needs_layout_passes=False, use_tc_tiling_on_sc=False),
      scratch_types=(
          [pltpu.VMEM((bsz,), I32)] * 4 + [pltpu.VMEM((bsz,), I32)] * 4
          + [pltpu.VMEM((bsz,), F32)] * 4 + [pltpu.VMEM((bsz, 32), F32)] * 4
          + [pltpu.VMEM_SHARED((npad, 32), F32)]
          + [pltpu.SemaphoreType.DMA] * 12
      ),
      interpret=interpret,
  )
  def k(idx16_h, dst_h, ext_h, h2d_h, z32_h, u_h, *refs):
    didv = refs[0:4]
    idxv = refs[4:8]
    exv = refs[8:12]
    hbuf = refs[12:16]
    acc = refs[16]
    asem = refs[17:21]
    gsem = refs[21:25]
    ssem = refs[25:29]
    c = lax.axis_index("c")
    s = lax.axis_index("s")

    @pl.loop(0, npass)
    def _(p):
      j = c * npass + p
      head = j // 2
      pltpu.sync_copy(z32_h.at[pl.ds(s * srows, srows)],
                      acc.at[pl.ds(s * srows, srows)])
      plsc.subcore_barrier()

      for b in range(4):
        off = s * chunk + b * bsz
        pltpu.async_copy(idx16_h.at[j, pl.ds(off, bsz)], idxv[b], asem[b])
        pltpu.async_copy(ext_h.at[head, pl.ds(off, bsz)], exv[b], asem[b])
        pltpu.async_copy(dst_h.at[pl.ds(off, bsz)], didv[b], asem[b])

      @pl.loop(0, nbatch // 4)
      def _(m):
        gd = [None] * 4
        for b in range(4):
          pltpu.make_async_copy(idx16_h.at[j, pl.ds(0, bsz)], idxv[b],
                                asem[b]).wait()
          pltpu.make_async_copy(ext_h.at[head, pl.ds(0, bsz)], exv[b],
                                asem[b]).wait()
          pltpu.make_async_copy(dst_h.at[pl.ds(0, bsz)], didv[b],
                                asem[b]).wait()
          gd[b] = pltpu.async_copy(h2d_h.at[idxv[b]], hbuf[b], gsem[b])
        more = m < nbatch // 4 - 1
        noff = s * chunk + (m + 1) * 4 * bsz
        sd = [None] * 4
        for b in range(4):
          gd[b].wait()

          @pl.when(more)
          def _(b=b):
            pltpu.async_copy(idx16_h.at[j, pl.ds(noff + b * bsz, bsz)],
                             idxv[b], asem[b])

          hb = hbuf[b]
          exb = exv[b]
          for g in range(bsz // LANES):
            evec = exb[pl.ds(g * LANES, LANES)]
            for rr in range(LANES):
              r = g * LANES + rr
              ev = evec.at[jnp.full((LANES,), rr, I32)].get(
                  mode="promise_in_bounds")
              hb[r, 0:16] = hb[r, 0:16] * ev
              hb[r, 16:32] = hb[r, 16:32] * ev

          @pl.when(more)
          def _(b=b):
            pltpu.async_copy(ext_h.at[head, pl.ds(noff + b * bsz, bsz)],
                             exv[b], asem[b])

          sd[b] = pltpu.async_copy(hb, acc.at[didv[b]], ssem[b], add=True)
        for b in range(4):
          sd[b].wait()

          @pl.when(more)
          def _(b=b):
            pltpu.async_copy(dst_h.at[pl.ds(noff + b * bsz, bsz)], didv[b],
                             asem[b])

      plsc.subcore_barrier()
      pltpu.sync_copy(acc.at[pl.ds(s * srows, srows)],
                      u_h.at[j, pl.ds(s * srows, srows)])
      plsc.subcore_barrier()

  return k(idx16, dst, ext, h2d, z32)


def _tcc(U, denp, b1r, W2r, *, n, npad, rblk, heads, interpret):
  nblk = npad // rblk

  def body(u_ref, dp_ref, b1_ref, w2_ref, h2_ref, mm_ref):
    den = dp_ref[0, :, 0:heads] + dp_ref[1, :, 0:heads] + 1e-16
    acc = jnp.zeros((rblk, 1), F32)
    for j in range(16):
      u = u_ref[j]
      dj = den[:, j // 2][:, None]
      hj = u / dj + b1_ref[j][None, :]
      hj = jnp.where(hj > 0, hj, jnp.exp(hj) - 1.0)
      acc = acc + jnp.dot(hj, w2_ref[j][:, None], preferred_element_type=F32)
    h2_ref[...] = acc
    rows = pl.program_id(0) * rblk + lax.broadcasted_iota(I32, (rblk, 1), 0)
    valid = rows < n
    hx = jnp.max(jnp.where(valid, acc, -jnp.inf)).reshape(1, 1)
    hn = jnp.min(jnp.where(valid, acc, jnp.inf)).reshape(1, 1)
    bm = jnp.concatenate([hn, hx], axis=1)

    @pl.when(pl.program_id(0) == 0)
    def _():
      mm_ref[...] = bm

    @pl.when(pl.program_id(0) > 0)
    def _():
      prev = mm_ref[...]
      mm_ref[...] = jnp.concatenate(
          [jnp.minimum(prev[:, 0:1], bm[:, 0:1]),
           jnp.maximum(prev[:, 1:2], bm[:, 1:2])], axis=1)

  return pl.pallas_call(
      body,
      grid=(nblk,),
      in_specs=[
          pl.BlockSpec((16, rblk, 32), lambda i: (0, i, 0)),
          pl.BlockSpec((NC, rblk, LANES), lambda i: (0, i, 0)),
          pl.BlockSpec((16, 32), lambda i: (0, 0)),
          pl.BlockSpec((16, 32), lambda i: (0, 0)),
      ],
      out_specs=[
          pl.BlockSpec((rblk, 1), lambda i: (i, 0)),
          pl.BlockSpec((1, 2), lambda i: (0, 0)),
      ],
      out_shape=[_sds((npad, 1), F32), _sds((1, 2), F32)],
      interpret=interpret,
  )(U, denp, b1r, W2r)


def _scd(src, dst, h2flat, params, z16, *, npad, epad, bsz, interpret):
  nw = NC * NS
  chunk = epad // nw
  nbatch = chunk // bsz
  srows = npad // NS
  mesh = plsc.VectorSubcoreMesh(
      core_axis_name="c", subcore_axis_name="s", num_cores=NC, num_subcores=NS)

  @functools.partial(
      pl.kernel,
      out_type=_sds((NC, npad, LANES), F32),
      mesh=mesh,
      compiler_params=pltpu.CompilerParams(needs_layout_passes=False, use_tc_tiling_on_sc=False),
      scratch_types=[
          pltpu.VMEM((npad,), F32),
          pltpu.VMEM((bsz,), I32),
          pltpu.VMEM((bsz,), I32),
          pltpu.VMEM((bsz, LANES), F32),
          pltpu.VMEM((LANES,), F32),
          pltpu.VMEM_SHARED((npad, LANES), F32),
      ],
      interpret=interpret,
  )
  def k(src_h, dst_h, h2_h, par_h, z16_h, out_h,
        h2v, sidv, didv, stage, pv, acc):
    c = lax.axis_index("c")
    s = lax.axis_index("s")
    w = s * NC + c
    pltpu.sync_copy(h2_h, h2v)
    pltpu.sync_copy(par_h, pv)
    pltpu.sync_copy(z16_h.at[pl.ds(s * srows, srows)],
                    acc.at[pl.ds(s * srows, srows)])
    plsc.subcore_barrier()
    lane = jax.lax.iota(I32, LANES)
    pvv = pv[...]
    take = lambda v, i: v.at[jnp.full((LANES,), i, I32)].get(
        mode="promise_in_bounds")
    cs = take(pvv, 0)
    cd = take(pvv, 1)
    mnv = take(pvv, 2)
    mxv = take(pvv, 3)
    s2max = jnp.maximum(cs * mxv, cs * mnv)
    zv = jnp.zeros((LANES,), F32)

    @pl.loop(0, nbatch)
    def _(t):
      off = w * chunk + t * bsz
      pltpu.sync_copy(src_h.at[pl.ds(off, bsz)], sidv)
      pltpu.sync_copy(dst_h.at[pl.ds(off, bsz)], didv)
      for g in range(bsz // LANES):
        sl = pl.ds(g * LANES, LANES)
        hs = plsc.load_gather(h2v, [sidv[sl]])
        hd = plsc.load_gather(h2v, [didv[sl]])
        al = _lrelu(cs * hs + cd * hd)
        cb = _lrelu(s2max + cd * hd)
        ev = jnp.exp(al - cb)
        val = ev * hs
        for r in range(LANES):
          vs = take(val, r)
          es = take(ev, r)
          row = jnp.where(lane == 0, vs, jnp.where(lane == 1, es, zv))
          stage[g * LANES + r, :] = row
      pltpu.sync_copy(stage, acc.at[didv], add=True)

    plsc.subcore_barrier()
    pltpu.sync_copy(acc.at[pl.ds(s * srows, srows)],
                    out_h.at[c, pl.ds(s * srows, srows)])

  return k(src, dst, h2flat, params, z16)


def _tce(out2, b2c, *, npad, rblk, interpret):
  nblk = npad // rblk

  def body(o_ref, b_ref, out_ref):
    o = o_ref[0] + o_ref[1]
    out_ref[...] = o[:, 0:1] / (o[:, 1:2] + 1e-16) + b_ref[...]

  return pl.pallas_call(
      body,
      grid=(nblk,),
      in_specs=[
          pl.BlockSpec((NC, rblk, LANES), lambda i: (0, i, 0)),
          pl.BlockSpec((1, 1), lambda i: (0, 0)),
      ],
      out_specs=pl.BlockSpec((rblk, 1), lambda i: (i, 0)),
      out_shape=_sds((npad, 1), F32),
      interpret=interpret,
  )(out2, b2c)


def _pipeline(x, edge_index, W1, att_src1, att_dst1, b1, W2, att_src2,
              att_dst2, b2, *, npad, epad, rblk, bsz, interpret=False):
  n, in_ch = x.shape
  heads, hid = att_src1.shape
  hidtot = heads * hid

  ei = edge_index.astype(I32)
  loop = jnp.arange(n, dtype=I32)
  e1 = ei.shape[1] + n
  src = jnp.concatenate(
      [ei[0], loop, jnp.zeros((epad - e1,), I32)])
  dst = jnp.concatenate(
      [ei[1], loop, jnp.full((epad - e1,), n, I32)])
  xp = jnp.pad(x, ((0, npad - n), (0, 0)))
  Asrc = jnp.pad((jnp.eye(heads, dtype=F32)[:, None, :]
                  * att_src1[:, :, None]).reshape(hidtot, heads),
                 ((0, 0), (0, LANES - heads)))
  Adst = jnp.pad((jnp.eye(heads, dtype=F32)[:, None, :]
                  * att_dst1[:, :, None]).reshape(hidtot, heads),
                 ((0, 0), (0, LANES - heads)))
  z16 = jnp.zeros((npad, LANES), F32)
  z32 = jnp.zeros((npad, 32), F32)

  h, asrc, adst, smax = _tc1(
      xp, W1, Asrc, Adst, npad=npad, rblk=rblk, heads=heads,
      interpret=interpret)
  ext, denp = _sca(
      src, dst, asrc, adst, smax.reshape(-1), z16, npad=npad, epad=epad,
      bsz=bsz, heads=heads, interpret=interpret)
  h2d = h.reshape(npad * 16, 32)
  idx16 = src[None, :] * 16 + jnp.arange(16, dtype=I32)[:, None]
  U = _scb(idx16, dst, ext, h2d, z32, npad=npad, epad=epad, bsz=bsz,
           interpret=interpret)
  b1r = b1.reshape(16, 32)
  W2r = W2[:, 0].reshape(16, 32)
  h2col, mm = _tcc(U, denp, b1r, W2r, n=n, npad=npad, rblk=rblk, heads=heads,
                   interpret=interpret)
  params = jnp.concatenate(
      [att_src2.reshape(-1)[:1], att_dst2.reshape(-1)[:1], mm[0],
       jnp.zeros((LANES - 4,), F32)])
  out2 = _scd(src, dst, h2col.reshape(npad), params, z16, npad=npad,
              epad=epad, bsz=bsz, interpret=interpret)
  outp = _tce(out2, b2.reshape(1, 1), npad=npad, rblk=rblk,
              interpret=interpret)
  return outp[:n]


def kernel(x, edge_index, W1, att_src1, att_dst1, b1, W2, att_src2, att_dst2,
           b2):
  return _pipeline(
      x, edge_index, W1, att_src1, att_dst1, b1, W2, att_src2, att_dst2, b2,
      npad=50176, epad=851968, rblk=1024, bsz=128)


# speedup vs baseline: 40.9700x; 1.0740x over previous
import functools

import jax
import jax.numpy as jnp
from jax import lax
from jax.experimental import pallas as pl
from jax.experimental.pallas import tpu as pltpu
from jax.experimental.pallas import tpu_sc as plsc

NC = 2
NS = 16
LANES = 16

F32 = jnp.float32
I32 = jnp.int32


def _lrelu(v):
  return jnp.where(v >= 0, v, 0.2 * v)


def _sds(shape, dtype):
  return jax.ShapeDtypeStruct(shape, dtype)


def _tc1(xp, W1, Asrc, Adst, *, npad, rblk, heads, interpret):
  nblk = npad // rblk
  in_ch = xp.shape[1]
  hidtot = W1.shape[1]

  def body(x_ref, w_ref, as_ref, ad_ref, h_ref, asrc_ref, adst_ref, smax_ref):
    h = jnp.dot(x_ref[...], w_ref[...], preferred_element_type=F32)
    h_ref[...] = h
    a_s = jnp.dot(h, as_ref[...], preferred_element_type=F32)
    a_d = jnp.dot(h, ad_ref[...], preferred_element_type=F32)
    asrc_ref[...] = a_s
    adst_ref[...] = a_d
    bm = jnp.max(a_s, axis=0, keepdims=True)

    @pl.when(pl.program_id(0) == 0)
    def _():
      smax_ref[...] = bm

    @pl.when(pl.program_id(0) > 0)
    def _():
      smax_ref[...] = jnp.maximum(smax_ref[...], bm)

  return pl.pallas_call(
      body,
      grid=(nblk,),
      in_specs=[
          pl.BlockSpec((rblk, in_ch), lambda i: (i, 0)),
          pl.BlockSpec((in_ch, hidtot), lambda i: (0, 0)),
          pl.BlockSpec((hidtot, LANES), lambda i: (0, 0)),
          pl.BlockSpec((hidtot, LANES), lambda i: (0, 0)),
      ],
      out_specs=[
          pl.BlockSpec((rblk, hidtot), lambda i: (i, 0)),
          pl.BlockSpec((rblk, LANES), lambda i: (i, 0)),
          pl.BlockSpec((rblk, LANES), lambda i: (i, 0)),
          pl.BlockSpec((1, LANES), lambda i: (0, 0)),
      ],
      out_shape=[
          _sds((npad, hidtot), F32),
          _sds((npad, LANES), F32),
          _sds((npad, LANES), F32),
          _sds((1, LANES), F32),
      ],
      interpret=interpret,
  )(xp, W1, Asrc, Adst)


def _sca(src, dst, asrc, adst, smax, z16, *, npad, epad, bsz, heads,
         interpret):
  nw = NC * NS
  chunk = epad // nw
  nbatch = chunk // bsz
  srows = npad // NS
  mesh = plsc.VectorSubcoreMesh(
      core_axis_name="c", subcore_axis_name="s", num_cores=NC, num_subcores=NS)

  @functools.partial(
      pl.kernel,
      out_type=(_sds((heads, epad), F32), _sds((NC, npad, LANES), F32)),
      mesh=mesh,
      compiler_params=pltpu.CompilerParams(needs_layout_passes=False, use_tc_tiling_on_sc=False),
      scratch_types=(
          [pltpu.VMEM((bsz,), I32)] * 4
          + [pltpu.VMEM((bsz, LANES), F32)] * 4
          + [pltpu.VMEM((bsz, LANES), F32),
             pltpu.VMEM((bsz * LANES,), F32),
             pltpu.VMEM((heads, bsz), F32),
             pltpu.VMEM((LANES,), F32),
             pltpu.VMEM_SHARED((npad, LANES), F32)]
          + [pltpu.SemaphoreType.DMA] * 6
      ),
      interpret=interpret,
  )
  def k(src_h, dst_h, asrc_h, adst_h, smax_h, z16_h, ext_h, den_h, *refs):
    sidv = refs[0:2]
    didv = refs[2:4]
    sbuf = refs[4:6]
    dbuf = refs[6:8]
    aos, aosf, soa, smv, acc = refs[8:13]
    isem = refs[13:15]
    gsem = refs[15:17]
    ssem = refs[17:19]
    c = lax.axis_index("c")
    s = lax.axis_index("s")
    w = s * NC + c
    pltpu.sync_copy(smax_h, smv)
    pltpu.sync_copy(z16_h.at[pl.ds(s * srows, srows)],
                    acc.at[pl.ds(s * srows, srows)])
    plsc.subcore_barrier()

    for b in range(2):
      off = w * chunk + b * bsz
      pltpu.async_copy(src_h.at[pl.ds(off, bsz)], sidv[b], isem[b])
      pltpu.async_copy(dst_h.at[pl.ds(off, bsz)], didv[b], isem[b])

    @pl.loop(0, nbatch // 2)
    def _(m):
      for b in range(2):
        t = m * 2 + b
        off = w * chunk + t * bsz
        pltpu.make_async_copy(src_h.at[pl.ds(0, bsz)], sidv[b],
                              isem[b]).wait()
        pltpu.make_async_copy(dst_h.at[pl.ds(0, bsz)], didv[b],
                              isem[b]).wait()
        d1 = pltpu.async_copy(asrc_h.at[sidv[b]], sbuf[b], gsem[b])
        d2 = pltpu.async_copy(adst_h.at[didv[b]], dbuf[b], gsem[b])
        if b == 0:
          continue
      smaxv = smv[...]
      more = m < nbatch // 2 - 1
      noff = w * chunk + (m + 1) * 2 * bsz
      sd = [None, None]
      for b in range(2):
        off = w * chunk + (m * 2 + b) * bsz
        pltpu.make_async_copy(asrc_h.at[sidv[b]], sbuf[b], gsem[b]).wait()
        pltpu.make_async_copy(adst_h.at[didv[b]], dbuf[b], gsem[b]).wait()

        @pl.when(more)
        def _(b=b):
          pltpu.async_copy(src_h.at[pl.ds(noff + b * bsz, bsz)], sidv[b],
                           isem[b])

        for r in range(bsz):
          sv = sbuf[b][r, :]
          dv = dbuf[b][r, :]
          al = _lrelu(sv + dv)
          cb = _lrelu(smaxv + dv)
          ev = jnp.exp(al - cb)
          aos[r, :] = ev
          aosf[pl.ds(r * LANES, LANES)] = ev
        lane = jax.lax.iota(I32, LANES)
        for g in range(bsz // LANES):
          ridx = (g * LANES + lane) * LANES
          for kk in range(heads):
            col = plsc.load_gather(aosf, [ridx + kk])
            soa[kk, pl.ds(g * LANES, LANES)] = col
        pltpu.sync_copy(soa, ext_h.at[:, pl.ds(off, bsz)])
        sd[b] = pltpu.async_copy(aos, acc.at[didv[b]], ssem[b], add=True)
        sd[b].wait()

        @pl.when(more)
        def _(b=b):
          pltpu.async_copy(dst_h.at[pl.ds(noff + b * bsz, bsz)], didv[b],
                           isem[b])

    plsc.subcore_barrier()
    pltpu.sync_copy(acc.at[pl.ds(s * srows, srows)],
                    den_h.at[c, pl.ds(s * srows, srows)])

  return k(src, dst, asrc, adst, smax, z16)


def _scb(idx16, dst, ext, h2d, z32, *, npad, epad, bsz, interpret):
  chunk = epad // NS
  nbatch = chunk // bsz
  srows = npad // NS
  npass = 8
  mesh = plsc.VectorSubcoreMesh(
      core_axis_name="c", subcore_axis_name="s", num_cores=NC, num_subcores=NS)

  @functools.partial(
      pl.kernel,
      out_type=_sds((2 * npass, npad, 32), F32),
      mesh=mesh,
      compiler_params=pltpu.CompilerParams(needs_layout_passes=False, use_tc_tiling_on_sc=False),
      scratch_types=(
          [pltpu.VMEM((bsz,), I32)] * 4 + [pltpu.VMEM((bsz,), I32)] * 4
          + [pltpu.VMEM((bsz,), F32)] * 4 + [pltpu.VMEM((bsz, 32), F32)] * 4
          + [pltpu.VMEM_SHARED((npad, 32), F32)]
          + [pltpu.SemaphoreType.DMA] * 12
      ),
      interpret=interpret,
  )
  def k(idx16_h, dst_h, ext_h, h2d_h, z32_h, u_h, *refs):
    didv = refs[0:4]
    idxv = refs[4:8]
    exv = refs[8:12]
    hbuf = refs[12:16]
    acc = refs[16]
    asem = refs[17:21]
    gsem = refs[21:25]
    ssem = refs[25:29]
    c = lax.axis_index("c")
    s = lax.axis_index("s")

    @pl.loop(0, npass)
    def _(p):
      j = c * npass + p
      head = j // 2
      pltpu.sync_copy(z32_h.at[pl.ds(s * srows, srows)],
                      acc.at[pl.ds(s * srows, srows)])
      plsc.subcore_barrier()

      for b in range(4):
        off = s * chunk + b * bsz
        pltpu.async_copy(idx16_h.at[j, pl.ds(off, bsz)], idxv[b], asem[b])
        pltpu.async_copy(ext_h.at[head, pl.ds(off, bsz)], exv[b], asem[b])
        pltpu.async_copy(dst_h.at[pl.ds(off, bsz)], didv[b], asem[b])

      @pl.loop(0, nbatch // 4)
      def _(m):
        gd = [None] * 4
        for b in range(4):
          pltpu.make_async_copy(idx16_h.at[j, pl.ds(0, bsz)], idxv[b],
                                asem[b]).wait()
          pltpu.make_async_copy(ext_h.at[head, pl.ds(0, bsz)], exv[b],
                                asem[b]).wait()
          pltpu.make_async_copy(dst_h.at[pl.ds(0, bsz)], didv[b],
                                asem[b]).wait()
          gd[b] = pltpu.async_copy(h2d_h.at[idxv[b]], hbuf[b], gsem[b])
        more = m < nbatch // 4 - 1
        noff = s * chunk + (m + 1) * 4 * bsz
        sd = [None] * 4
        for b in range(4):
          gd[b].wait()

          @pl.when(more)
          def _(b=b):
            pltpu.async_copy(idx16_h.at[j, pl.ds(noff + b * bsz, bsz)],
                             idxv[b], asem[b])

          hb = hbuf[b]
          exb = exv[b]
          for g in range(bsz // LANES):
            evec = exb[pl.ds(g * LANES, LANES)]
            for rr in range(LANES):
              r = g * LANES + rr
              ev = evec.at[jnp.full((LANES,), rr, I32)].get(
                  mode="promise_in_bounds")
              hb[r, 0:16] = hb[r, 0:16] * ev
              hb[r, 16:32] = hb[r, 16:32] * ev

          @pl.when(more)
          def _(b=b):
            pltpu.async_copy(ext_h.at[head, pl.ds(noff + b * bsz, bsz)],
                             exv[b], asem[b])

          sd[b] = pltpu.async_copy(hb, acc.at[didv[b]], ssem[b], add=True)
        for b in range(4):
          sd[b].wait()

          @pl.when(more)
          def _(b=b):
            pltpu.async_copy(dst_h.at[pl.ds(noff + b * bsz, bsz)], didv[b],
                             asem[b])

      plsc.subcore_barrier()
      pltpu.sync_copy(acc.at[pl.ds(s * srows, srows)],
                      u_h.at[j, pl.ds(s * srows, srows)])
      plsc.subcore_barrier()

  return k(idx16, dst, ext, h2d, z32)


def _tcc(U, denp, b1r, W2r, *, n, npad, rblk, heads, interpret):
  nblk = npad // rblk

  def body(u_ref, dp_ref, b1_ref, w2_ref, h2_ref, mm_ref):
    den = dp_ref[0, :, 0:heads] + dp_ref[1, :, 0:heads] + 1e-16
    acc = jnp.zeros((rblk, 1), F32)
    for j in range(16):
      u = u_ref[j]
      dj = den[:, j // 2][:, None]
      hj = u / dj + b1_ref[j][None, :]
      hj = jnp.where(hj > 0, hj, jnp.exp(hj) - 1.0)
      acc = acc + jnp.dot(hj, w2_ref[j][:, None], preferred_element_type=F32)
    h2_ref[...] = acc
    rows = pl.program_id(0) * rblk + lax.broadcasted_iota(I32, (rblk, 1), 0)
    valid = rows < n
    hx = jnp.max(jnp.where(valid, acc, -jnp.inf)).reshape(1, 1)
    hn = jnp.min(jnp.where(valid, acc, jnp.inf)).reshape(1, 1)
    bm = jnp.concatenate([hn, hx], axis=1)

    @pl.when(pl.program_id(0) == 0)
    def _():
      mm_ref[...] = bm

    @pl.when(pl.program_id(0) > 0)
    def _():
      prev = mm_ref[...]
      mm_ref[...] = jnp.concatenate(
          [jnp.minimum(prev[:, 0:1], bm[:, 0:1]),
           jnp.maximum(prev[:, 1:2], bm[:, 1:2])], axis=1)

  return pl.pallas_call(
      body,
      grid=(nblk,),
      in_specs=[
          pl.BlockSpec((16, rblk, 32), lambda i: (0, i, 0)),
          pl.BlockSpec((NC, rblk, LANES), lambda i: (0, i, 0)),
          pl.BlockSpec((16, 32), lambda i: (0, 0)),
          pl.BlockSpec((16, 32), lambda i: (0, 0)),
      ],
      out_specs=[
          pl.BlockSpec((rblk, 1), lambda i: (i, 0)),
          pl.BlockSpec((1, 2), lambda i: (0, 0)),
      ],
      out_shape=[_sds((npad, 1), F32), _sds((1, 2), F32)],
      interpret=interpret,
  )(U, denp, b1r, W2r)


def _scd(src, dst, h2flat, params, z16, *, npad, epad, bsz, interpret):
  nw = NC * NS
  chunk = epad // nw
  nbatch = chunk // bsz
  srows = npad // NS
  mesh = plsc.VectorSubcoreMesh(
      core_axis_name="c", subcore_axis_name="s", num_cores=NC, num_subcores=NS)

  @functools.partial(
      pl.kernel,
      out_type=_sds((NC, npad, LANES), F32),
      mesh=mesh,
      compiler_params=pltpu.CompilerParams(needs_layout_passes=False, use_tc_tiling_on_sc=False),
      scratch_types=[
          pltpu.VMEM((npad,), F32),
          pltpu.VMEM((bsz,), I32),
          pltpu.VMEM((bsz,), I32),
          pltpu.VMEM((bsz, LANES), F32),
          pltpu.VMEM((LANES,), F32),
          pltpu.VMEM_SHARED((npad, LANES), F32),
      ],
      interpret=interpret,
  )
  def k(src_h, dst_h, h2_h, par_h, z16_h, out_h,
        h2v, sidv, didv, stage, pv, acc):
    c = lax.axis_index("c")
    s = lax.axis_index("s")
    w = s * NC + c
    pltpu.sync_copy(h2_h, h2v)
    pltpu.sync_copy(par_h, pv)
    pltpu.sync_copy(z16_h.at[pl.ds(s * srows, srows)],
                    acc.at[pl.ds(s * srows, srows)])
    plsc.subcore_barrier()
    lane = jax.lax.iota(I32, LANES)
    pvv = pv[...]
    take = lambda v, i: v.at[jnp.full((LANES,), i, I32)].get(
        mode="promise_in_bounds")
    cs = take(pvv, 0)
    cd = take(pvv, 1)
    mnv = take(pvv, 2)
    mxv = take(pvv, 3)
    s2max = jnp.maximum(cs * mxv, cs * mnv)
    zv = jnp.zeros((LANES,), F32)

    @pl.loop(0, nbatch)
    def _(t):
      off = w * chunk + t * bsz
      pltpu.sync_copy(src_h.at[pl.ds(off, bsz)], sidv)
      pltpu.sync_copy(dst_h.at[pl.ds(off, bsz)], didv)
      for g in range(bsz // LANES):
        sl = pl.ds(g * LANES, LANES)
        hs = plsc.load_gather(h2v, [sidv[sl]])
        hd = plsc.load_gather(h2v, [didv[sl]])
        al = _lrelu(cs * hs + cd * hd)
        cb = _lrelu(s2max + cd * hd)
        ev = jnp.exp(al - cb)
        val = ev * hs
        for r in range(LANES):
          vs = take(val, r)
          es = take(ev, r)
          row = jnp.where(lane == 0, vs, jnp.where(lane == 1, es, zv))
          stage[g * LANES + r, :] = row
      pltpu.sync_copy(stage, acc.at[didv], add=True)

    plsc.subcore_barrier()
    pltpu.sync_copy(acc.at[pl.ds(s * srows, srows)],
                    out_h.at[c, pl.ds(s * srows, srows)])

  return k(src, dst, h2flat, params, z16)


def _tce(out2, b2c, *, npad, rblk, interpret):
  nblk = npad // rblk

  def body(o_ref, b_ref, out_ref):
    o = o_ref[0] + o_ref[1]
    out_ref[...] = o[:, 0:1] / (o[:, 1:2] + 1e-16) + b_ref[...]

  return pl.pallas_call(
      body,
      grid=(nblk,),
      in_specs=[
          pl.BlockSpec((NC, rblk, LANES), lambda i: (0, i, 0)),
          pl.BlockSpec((1, 1), lambda i: (0, 0)),
      ],
      out_specs=pl.BlockSpec((rblk, 1), lambda i: (i, 0)),
      out_shape=_sds((npad, 1), F32),
      interpret=interpret,
  )(out2, b2c)


def _pipeline(x, edge_index, W1, att_src1, att_dst1, b1, W2, att_src2,
              att_dst2, b2, *, npad, epad, rblk, bsz, interpret=False):
  n, in_ch = x.shape
  heads, hid = att_src1.shape
  hidtot = heads * hid

  ei = edge_index.astype(I32)
  loop = jnp.arange(n, dtype=I32)
  e1 = ei.shape[1] + n
  src = jnp.concatenate(
      [ei[0], loop, jnp.zeros((epad - e1,), I32)])
  dst = jnp.concatenate(
      [ei[1], loop, jnp.full((epad - e1,), n, I32)])
  xp = jnp.pad(x, ((0, npad - n), (0, 0)))
  Asrc = jnp.pad((jnp.eye(heads, dtype=F32)[:, None, :]
                  * att_src1[:, :, None]).reshape(hidtot, heads),
                 ((0, 0), (0, LANES - heads)))
  Adst = jnp.pad((jnp.eye(heads, dtype=F32)[:, None, :]
                  * att_dst1[:, :, None]).reshape(hidtot, heads),
                 ((0, 0), (0, LANES - heads)))
  z16 = jnp.zeros((npad, LANES), F32)
  z32 = jnp.zeros((npad, 32), F32)

  h, asrc, adst, smax = _tc1(
      xp, W1, Asrc, Adst, npad=npad, rblk=rblk, heads=heads,
      interpret=interpret)
  ext, denp = _sca(
      src, dst, asrc, adst, smax.reshape(-1), z16, npad=npad, epad=epad,
      bsz=bsz, heads=heads, interpret=interpret)
  h2d = h.reshape(npad * 16, 32)
  idx16 = src[None, :] * 16 + jnp.arange(16, dtype=I32)[:, None]
  U = _scb(idx16, dst, ext, h2d, z32, npad=npad, epad=epad, bsz=bsz,
           interpret=interpret)
  b1r = b1.reshape(16, 32)
  W2r = W2[:, 0].reshape(16, 32)
  h2col, mm = _tcc(U, denp, b1r, W2r, n=n, npad=npad, rblk=rblk, heads=heads,
                   interpret=interpret)
  params = jnp.concatenate(
      [att_src2.reshape(-1)[:1], att_dst2.reshape(-1)[:1], mm[0],
       jnp.zeros((LANES - 4,), F32)])
  out2 = _scd(src, dst, h2col.reshape(npad), params, z16, npad=npad,
              epad=epad, bsz=bsz, interpret=interpret)
  outp = _tce(out2, b2.reshape(1, 1), npad=npad, rblk=rblk,
              interpret=interpret)
  return outp[:n]


def kernel(x, edge_index, W1, att_src1, att_dst1, b1, W2, att_src2, att_dst2,
           b2):
  return _pipeline(
      x, edge_index, W1, att_src1, att_dst1, b1, W2, att_src2, att_dst2, b2,
      npad=50176, epad=851968, rblk=1024, bsz=128)
